# Initial kernel scaffold; baseline (speedup 1.0000x reference)
import jax
import jax.numpy as jnp
from jax.experimental import pallas as pl


def kernel(x, edge_attr, edge_index, shapes, we1, be1, we2, be2, ge, bbe, wn1, bn1, wn2, bn2, gn, bbn):
    raise NotImplementedError("write your pallas kernel here")


# speedup vs baseline: 1.1389x; 1.1389x over previous
import functools

import jax
import jax.numpy as jnp
from jax.experimental import pallas as pl
from jax.experimental.pallas import tpu as pltpu

N = 10000
E = 320000
D = 128

ROW_BLK_PRE = 2000
ROW_BLK_EDGE = 1280
ROW_BLK_NODE = 2000


def _pre_body(x_ref, wi_ref, wj_ref, px_ref, qx_ref):
    x = x_ref[...]
    px_ref[...] = jnp.dot(x, wi_ref[...], preferred_element_type=jnp.float32)
    qx_ref[...] = jnp.dot(x, wj_ref[...], preferred_element_type=jnp.float32)


def _precompute(x, wi, wj):
    grid = (N // ROW_BLK_PRE,)
    return pl.pallas_call(
        _pre_body,
        grid=grid,
        in_specs=[
            pl.BlockSpec((ROW_BLK_PRE, D), lambda i: (i, 0)),
            pl.BlockSpec((D, D), lambda i: (0, 0)),
            pl.BlockSpec((D, D), lambda i: (0, 0)),
        ],
        out_specs=[
            pl.BlockSpec((ROW_BLK_PRE, D), lambda i: (i, 0)),
            pl.BlockSpec((ROW_BLK_PRE, D), lambda i: (i, 0)),
        ],
        out_shape=[
            jax.ShapeDtypeStruct((N, D), jnp.float32),
            jax.ShapeDtypeStruct((N, D), jnp.float32),
        ],
    )(x, wi, wj)


def _edge_body(g_ref, ea_ref, we_ref, b1_ref, w2_ref, b2_ref, ge_ref, bbe_ref,
               out_ref):
    ea = ea_ref[...]
    h = g_ref[...] + jnp.dot(ea, we_ref[...],
                             preferred_element_type=jnp.float32) + b1_ref[...]
    h = h * jax.nn.sigmoid(h)
    h = jnp.dot(h, w2_ref[...], preferred_element_type=jnp.float32) + b2_ref[...]
    mu = jnp.mean(h, axis=-1, keepdims=True)
    var = jnp.mean((h - mu) * (h - mu), axis=-1, keepdims=True)
    h = (h - mu) * jax.lax.rsqrt(var + 1e-5) * ge_ref[...] + bbe_ref[...]
    out_ref[...] = h + ea


def _edge_mlp(g, ea, we, b1, w2, b2, ge, bbe):
    grid = (E // ROW_BLK_EDGE,)
    row = lambda i: (i, 0)
    full = lambda i: (0, 0)
    return pl.pallas_call(
        _edge_body,
        grid=grid,
        in_specs=[
            pl.BlockSpec((ROW_BLK_EDGE, D), row),
            pl.BlockSpec((ROW_BLK_EDGE, D), row),
            pl.BlockSpec((D, D), full),
            pl.BlockSpec((1, D), full),
            pl.BlockSpec((D, D), full),
            pl.BlockSpec((1, D), full),
            pl.BlockSpec((1, D), full),
            pl.BlockSpec((1, D), full),
        ],
        out_specs=pl.BlockSpec((ROW_BLK_EDGE, D), row),
        out_shape=jax.ShapeDtypeStruct((E, D), jnp.float32),
    )(g, ea, we, b1, w2, b2, ge, bbe)


def _node_body(x_ref, agg_ref, wa_ref, wb_ref, b1_ref, w2_ref, b2_ref,
               gn_ref, bbn_ref, out_ref):
    x = x_ref[...]
    h = (jnp.dot(x, wa_ref[...], preferred_element_type=jnp.float32)
         + jnp.dot(agg_ref[...], wb_ref[...], preferred_element_type=jnp.float32)
         + b1_ref[...])
    h = h * jax.nn.sigmoid(h)
    h = jnp.dot(h, w2_ref[...], preferred_element_type=jnp.float32) + b2_ref[...]
    mu = jnp.mean(h, axis=-1, keepdims=True)
    var = jnp.mean((h - mu) * (h - mu), axis=-1, keepdims=True)
    h = (h - mu) * jax.lax.rsqrt(var + 1e-5) * gn_ref[...] + bbn_ref[...]
    out_ref[...] = h + x


def _node_mlp(x, agg, wa, wb, b1, w2, b2, gn, bbn):
    grid = (N // ROW_BLK_NODE,)
    row = lambda i: (i, 0)
    full = lambda i: (0, 0)
    return pl.pallas_call(
        _node_body,
        grid=grid,
        in_specs=[
            pl.BlockSpec((ROW_BLK_NODE, D), row),
            pl.BlockSpec((ROW_BLK_NODE, D), row),
            pl.BlockSpec((D, D), full),
            pl.BlockSpec((D, D), full),
            pl.BlockSpec((1, D), full),
            pl.BlockSpec((D, D), full),
            pl.BlockSpec((1, D), full),
            pl.BlockSpec((1, D), full),
            pl.BlockSpec((1, D), full),
        ],
        out_specs=pl.BlockSpec((ROW_BLK_NODE, D), row),
        out_shape=jax.ShapeDtypeStruct((N, D), jnp.float32),
    )(x, agg, wa, wb, b1, w2, b2, gn, bbn)


def kernel(x, edge_attr, edge_index, shapes, we1, be1, we2, be2, ge, bbe,
           wn1, bn1, wn2, bn2, gn, bbn):
    del shapes
    src = edge_index[0]
    dst = edge_index[1]
    wi, wj, we = we1[:D], we1[D:2 * D], we1[2 * D:]
    px, qx = _precompute(x, wi, wj)
    g = jnp.take(px, dst, axis=0) + jnp.take(qx, src, axis=0)
    edges_new = _edge_mlp(g, edge_attr, we, be1[None, :], we2, be2[None, :],
                          ge[None, :], bbe[None, :])
    agg = jax.ops.segment_sum(edges_new, dst, num_segments=N)
    nodes_new = _node_mlp(x, agg, wn1[:D], wn1[D:], bn1[None, :], wn2,
                          bn2[None, :], gn[None, :], bbn[None, :])
    return nodes_new, edges_new


# speedup vs baseline: 2.0597x; 1.8085x over previous
import functools

import jax
import jax.numpy as jnp
from jax import lax
from jax.experimental import pallas as pl
from jax.experimental.pallas import tpu as pltpu
from jax.experimental.pallas import tpu_sc as plsc

N = 10000
E = 320000
D = 128

SC_NC = 2
SC_NS = 16
SC_NW = SC_NC * SC_NS
E_PER_W = E // SC_NW
GCHUNK = 80
GSTEPS = E_PER_W // GCHUNK

ROW_BLK_PRE = 2000
ROW_BLK_EDGE = 1280
ROW_BLK_NODE = 2000


def _pre_body(x_ref, wi_ref, wj_ref, px_ref, qx_ref):
    x = x_ref[...]
    px_ref[...] = jnp.dot(x, wi_ref[...], preferred_element_type=jnp.float32)
    qx_ref[...] = jnp.dot(x, wj_ref[...], preferred_element_type=jnp.float32)


def _precompute(x, wi, wj):
    grid = (N // ROW_BLK_PRE,)
    return pl.pallas_call(
        _pre_body,
        grid=grid,
        in_specs=[
            pl.BlockSpec((ROW_BLK_PRE, D), lambda i: (i, 0)),
            pl.BlockSpec((D, D), lambda i: (0, 0)),
            pl.BlockSpec((D, D), lambda i: (0, 0)),
        ],
        out_specs=[
            pl.BlockSpec((ROW_BLK_PRE, D), lambda i: (i, 0)),
            pl.BlockSpec((ROW_BLK_PRE, D), lambda i: (i, 0)),
        ],
        out_shape=[
            jax.ShapeDtypeStruct((N, D), jnp.float32),
            jax.ShapeDtypeStruct((N, D), jnp.float32),
        ],
    )(x, wi, wj)


def _sc_gather_body(px_hbm, qx_hbm, dst_hbm, src_hbm, gi_hbm, gj_hbm,
                    di_v, si_v, rows_a, rows_b, sem_a, sem_b):
    wid = lax.axis_index("c") * SC_NS + lax.axis_index("s")
    wbase = wid * E_PER_W

    def step(it, _):
        base = wbase + it * GCHUNK
        pltpu.sync_copy(dst_hbm.at[pl.ds(base, GCHUNK)], di_v)
        pltpu.sync_copy(src_hbm.at[pl.ds(base, GCHUNK)], si_v)
        cp_a = pltpu.async_copy(px_hbm.at[di_v], rows_a, sem_a)
        cp_b = pltpu.async_copy(qx_hbm.at[si_v], rows_b, sem_b)
        cp_a.wait()
        cp_b.wait()
        pltpu.sync_copy(rows_a, gi_hbm.at[pl.ds(base, GCHUNK)])
        pltpu.sync_copy(rows_b, gj_hbm.at[pl.ds(base, GCHUNK)])
        return ()

    lax.fori_loop(0, GSTEPS, step, ())


@functools.partial(
    pl.kernel,
    out_type=[
        jax.ShapeDtypeStruct((E, D), jnp.float32),
        jax.ShapeDtypeStruct((E, D), jnp.float32),
    ],
    mesh=plsc.VectorSubcoreMesh(core_axis_name="c", subcore_axis_name="s"),
    scratch_types=[
        pltpu.VMEM((GCHUNK,), jnp.int32),
        pltpu.VMEM((GCHUNK,), jnp.int32),
        pltpu.VMEM((GCHUNK, D), jnp.float32),
        pltpu.VMEM((GCHUNK, D), jnp.float32),
        pltpu.SemaphoreType.DMA,
        pltpu.SemaphoreType.DMA,
    ],
)
def _sc_gather(px_hbm, qx_hbm, dst_hbm, src_hbm, gi_hbm, gj_hbm,
               di_v, si_v, rows_a, rows_b, sem_a, sem_b):
    _sc_gather_body(px_hbm, qx_hbm, dst_hbm, src_hbm, gi_hbm, gj_hbm,
                    di_v, si_v, rows_a, rows_b, sem_a, sem_b)


def _edge_body(gi_ref, gj_ref, ea_ref, we_ref, b1_ref, w2_ref, b2_ref, ge_ref,
               bbe_ref, out_ref):
    ea = ea_ref[...]
    h = gi_ref[...] + gj_ref[...] + jnp.dot(
        ea, we_ref[...], preferred_element_type=jnp.float32) + b1_ref[...]
    h = h * jax.nn.sigmoid(h)
    h = jnp.dot(h, w2_ref[...], preferred_element_type=jnp.float32) + b2_ref[...]
    mu = jnp.mean(h, axis=-1, keepdims=True)
    var = jnp.mean((h - mu) * (h - mu), axis=-1, keepdims=True)
    h = (h - mu) * jax.lax.rsqrt(var + 1e-5) * ge_ref[...] + bbe_ref[...]
    out_ref[...] = h + ea


def _edge_mlp(gi, gj, ea, we, b1, w2, b2, ge, bbe):
    grid = (E // ROW_BLK_EDGE,)
    row = lambda i: (i, 0)
    full = lambda i: (0, 0)
    return pl.pallas_call(
        _edge_body,
        grid=grid,
        in_specs=[
            pl.BlockSpec((ROW_BLK_EDGE, D), row),
            pl.BlockSpec((ROW_BLK_EDGE, D), row),
            pl.BlockSpec((ROW_BLK_EDGE, D), row),
            pl.BlockSpec((D, D), full),
            pl.BlockSpec((1, D), full),
            pl.BlockSpec((D, D), full),
            pl.BlockSpec((1, D), full),
            pl.BlockSpec((1, D), full),
            pl.BlockSpec((1, D), full),
        ],
        out_specs=pl.BlockSpec((ROW_BLK_EDGE, D), row),
        out_shape=jax.ShapeDtypeStruct((E, D), jnp.float32),
    )(gi, gj, ea, we, b1, w2, b2, ge, bbe)


def _node_body(x_ref, agg_ref, wa_ref, wb_ref, b1_ref, w2_ref, b2_ref,
               gn_ref, bbn_ref, out_ref):
    x = x_ref[...]
    h = (jnp.dot(x, wa_ref[...], preferred_element_type=jnp.float32)
         + jnp.dot(agg_ref[...], wb_ref[...], preferred_element_type=jnp.float32)
         + b1_ref[...])
    h = h * jax.nn.sigmoid(h)
    h = jnp.dot(h, w2_ref[...], preferred_element_type=jnp.float32) + b2_ref[...]
    mu = jnp.mean(h, axis=-1, keepdims=True)
    var = jnp.mean((h - mu) * (h - mu), axis=-1, keepdims=True)
    h = (h - mu) * jax.lax.rsqrt(var + 1e-5) * gn_ref[...] + bbn_ref[...]
    out_ref[...] = h + x


def _node_mlp(x, agg, wa, wb, b1, w2, b2, gn, bbn):
    grid = (N // ROW_BLK_NODE,)
    row = lambda i: (i, 0)
    full = lambda i: (0, 0)
    return pl.pallas_call(
        _node_body,
        grid=grid,
        in_specs=[
            pl.BlockSpec((ROW_BLK_NODE, D), row),
            pl.BlockSpec((ROW_BLK_NODE, D), row),
            pl.BlockSpec((D, D), full),
            pl.BlockSpec((D, D), full),
            pl.BlockSpec((1, D), full),
            pl.BlockSpec((D, D), full),
            pl.BlockSpec((1, D), full),
            pl.BlockSpec((1, D), full),
            pl.BlockSpec((1, D), full),
        ],
        out_specs=pl.BlockSpec((ROW_BLK_NODE, D), row),
        out_shape=jax.ShapeDtypeStruct((N, D), jnp.float32),
    )(x, agg, wa, wb, b1, w2, b2, gn, bbn)


def kernel(x, edge_attr, edge_index, shapes, we1, be1, we2, be2, ge, bbe,
           wn1, bn1, wn2, bn2, gn, bbn):
    del shapes
    src = edge_index[0]
    dst = edge_index[1]
    wi, wj, we = we1[:D], we1[D:2 * D], we1[2 * D:]
    px, qx = _precompute(x, wi, wj)
    gi, gj = _sc_gather(px, qx, dst, src)
    edges_new = _edge_mlp(gi, gj, edge_attr, we, be1[None, :], we2, be2[None, :],
                          ge[None, :], bbe[None, :])
    agg = jax.ops.segment_sum(edges_new, dst, num_segments=N)
    nodes_new = _node_mlp(x, agg, wn1[:D], wn1[D:], bn1[None, :], wn2,
                          bn2[None, :], gn[None, :], bbn[None, :])
    return nodes_new, edges_new


# speedup vs baseline: 2.9220x; 1.4186x over previous
import functools

import jax
import jax.numpy as jnp
from jax import lax
from jax.experimental import pallas as pl
from jax.experimental.pallas import tpu as pltpu
from jax.experimental.pallas import tpu_sc as plsc

N = 10000
E = 320000
D = 128

SC_NC = 2
SC_NS = 16
SC_NW = SC_NC * SC_NS
E_PER_W = E // SC_NW
GCHUNK = 80
GSTEPS = E_PER_W // GCHUNK

ROW_BLK_PRE = 2000
ROW_BLK_EDGE = 1280
ROW_BLK_NODE = 2000


def _pre_body(x_ref, wi_ref, wj_ref, px_ref, qx_ref):
    x = x_ref[...]
    px_ref[...] = jnp.dot(x, wi_ref[...], preferred_element_type=jnp.float32)
    qx_ref[...] = jnp.dot(x, wj_ref[...], preferred_element_type=jnp.float32)


def _precompute(x, wi, wj):
    grid = (N // ROW_BLK_PRE,)
    return pl.pallas_call(
        _pre_body,
        grid=grid,
        in_specs=[
            pl.BlockSpec((ROW_BLK_PRE, D), lambda i: (i, 0)),
            pl.BlockSpec((D, D), lambda i: (0, 0)),
            pl.BlockSpec((D, D), lambda i: (0, 0)),
        ],
        out_specs=[
            pl.BlockSpec((ROW_BLK_PRE, D), lambda i: (i, 0)),
            pl.BlockSpec((ROW_BLK_PRE, D), lambda i: (i, 0)),
        ],
        out_shape=[
            jax.ShapeDtypeStruct((N, D), jnp.float32),
            jax.ShapeDtypeStruct((N, D), jnp.float32),
        ],
    )(x, wi, wj)


def _sc_gather_body(px_hbm, qx_hbm, dst_hbm, src_hbm, gi_hbm, gj_hbm,
                    di_v, si_v, rows_a, rows_b, sem_a, sem_b):
    wid = lax.axis_index("c") * SC_NS + lax.axis_index("s")
    wbase = wid * E_PER_W

    def step(it, _):
        base = wbase + it * GCHUNK
        pltpu.sync_copy(dst_hbm.at[pl.ds(base, GCHUNK)], di_v)
        pltpu.sync_copy(src_hbm.at[pl.ds(base, GCHUNK)], si_v)
        cp_a = pltpu.async_copy(px_hbm.at[di_v], rows_a, sem_a)
        cp_b = pltpu.async_copy(qx_hbm.at[si_v], rows_b, sem_b)
        cp_a.wait()
        cp_b.wait()
        pltpu.sync_copy(rows_a, gi_hbm.at[pl.ds(base, GCHUNK)])
        pltpu.sync_copy(rows_b, gj_hbm.at[pl.ds(base, GCHUNK)])
        return ()

    lax.fori_loop(0, GSTEPS, step, ())


@functools.partial(
    pl.kernel,
    out_type=[
        jax.ShapeDtypeStruct((E, D), jnp.float32),
        jax.ShapeDtypeStruct((E, D), jnp.float32),
    ],
    mesh=plsc.VectorSubcoreMesh(core_axis_name="c", subcore_axis_name="s"),
    scratch_types=[
        pltpu.VMEM((GCHUNK,), jnp.int32),
        pltpu.VMEM((GCHUNK,), jnp.int32),
        pltpu.VMEM((GCHUNK, D), jnp.float32),
        pltpu.VMEM((GCHUNK, D), jnp.float32),
        pltpu.SemaphoreType.DMA,
        pltpu.SemaphoreType.DMA,
    ],
)
def _sc_gather(px_hbm, qx_hbm, dst_hbm, src_hbm, gi_hbm, gj_hbm,
               di_v, si_v, rows_a, rows_b, sem_a, sem_b):
    _sc_gather_body(px_hbm, qx_hbm, dst_hbm, src_hbm, gi_hbm, gj_hbm,
                    di_v, si_v, rows_a, rows_b, sem_a, sem_b)


NPAD = 10240
NROWS_PER_TILE = NPAD // SC_NS
SCHUNK = 80
SSTEPS = E_PER_W // SCHUNK


def _sc_scatter_body(edges_hbm, dst_hbm, zs_hbm, out_hbm, idx_v, rows_v, acc_sh,
                     sem):
    c = lax.axis_index("c")
    s = lax.axis_index("s")
    wid = c * SC_NS + s
    wbase = wid * E_PER_W
    pltpu.sync_copy(zs_hbm, acc_sh.at[pl.ds(s * NROWS_PER_TILE, NROWS_PER_TILE)])
    plsc.subcore_barrier()

    def step(it, _):
        base = wbase + it * SCHUNK
        pltpu.sync_copy(dst_hbm.at[pl.ds(base, SCHUNK)], idx_v)
        pltpu.sync_copy(edges_hbm.at[pl.ds(base, SCHUNK)], rows_v)
        pltpu.sync_copy(rows_v, acc_sh.at[idx_v], add=True)
        return ()

    lax.fori_loop(0, SSTEPS, step, ())
    plsc.subcore_barrier()
    pltpu.sync_copy(acc_sh.at[pl.ds(s * NROWS_PER_TILE, NROWS_PER_TILE)],
                    out_hbm.at[pl.ds(c * NPAD + s * NROWS_PER_TILE,
                                     NROWS_PER_TILE)])


@functools.partial(
    pl.kernel,
    out_type=jax.ShapeDtypeStruct((2 * NPAD, D), jnp.float32),
    mesh=plsc.VectorSubcoreMesh(core_axis_name="c", subcore_axis_name="s"),
    scratch_types=[
        pltpu.VMEM((SCHUNK,), jnp.int32),
        pltpu.VMEM((SCHUNK, D), jnp.float32),
        pltpu.VMEM_SHARED((NPAD, D), jnp.float32),
        pltpu.SemaphoreType.DMA,
    ],
)
def _sc_scatter(edges_hbm, dst_hbm, zs_hbm, out_hbm, idx_v, rows_v, acc_sh, sem):
    _sc_scatter_body(edges_hbm, dst_hbm, zs_hbm, out_hbm, idx_v, rows_v, acc_sh,
                     sem)


def _edge_body(gi_ref, gj_ref, ea_ref, we_ref, b1_ref, w2_ref, b2_ref, ge_ref,
               bbe_ref, out_ref):
    ea = ea_ref[...]
    h = gi_ref[...] + gj_ref[...] + jnp.dot(
        ea, we_ref[...], preferred_element_type=jnp.float32) + b1_ref[...]
    h = h * jax.nn.sigmoid(h)
    h = jnp.dot(h, w2_ref[...], preferred_element_type=jnp.float32) + b2_ref[...]
    mu = jnp.mean(h, axis=-1, keepdims=True)
    var = jnp.mean((h - mu) * (h - mu), axis=-1, keepdims=True)
    h = (h - mu) * jax.lax.rsqrt(var + 1e-5) * ge_ref[...] + bbe_ref[...]
    out_ref[...] = h + ea


def _edge_mlp(gi, gj, ea, we, b1, w2, b2, ge, bbe):
    grid = (E // ROW_BLK_EDGE,)
    row = lambda i: (i, 0)
    full = lambda i: (0, 0)
    return pl.pallas_call(
        _edge_body,
        grid=grid,
        in_specs=[
            pl.BlockSpec((ROW_BLK_EDGE, D), row),
            pl.BlockSpec((ROW_BLK_EDGE, D), row),
            pl.BlockSpec((ROW_BLK_EDGE, D), row),
            pl.BlockSpec((D, D), full),
            pl.BlockSpec((1, D), full),
            pl.BlockSpec((D, D), full),
            pl.BlockSpec((1, D), full),
            pl.BlockSpec((1, D), full),
            pl.BlockSpec((1, D), full),
        ],
        out_specs=pl.BlockSpec((ROW_BLK_EDGE, D), row),
        out_shape=jax.ShapeDtypeStruct((E, D), jnp.float32),
    )(gi, gj, ea, we, b1, w2, b2, ge, bbe)


def _node_body(x_ref, p0_ref, p1_ref, wa_ref, wb_ref, b1_ref, w2_ref, b2_ref,
               gn_ref, bbn_ref, out_ref):
    x = x_ref[...]
    agg = p0_ref[...] + p1_ref[...]
    h = (jnp.dot(x, wa_ref[...], preferred_element_type=jnp.float32)
         + jnp.dot(agg, wb_ref[...], preferred_element_type=jnp.float32)
         + b1_ref[...])
    h = h * jax.nn.sigmoid(h)
    h = jnp.dot(h, w2_ref[...], preferred_element_type=jnp.float32) + b2_ref[...]
    mu = jnp.mean(h, axis=-1, keepdims=True)
    var = jnp.mean((h - mu) * (h - mu), axis=-1, keepdims=True)
    h = (h - mu) * jax.lax.rsqrt(var + 1e-5) * gn_ref[...] + bbn_ref[...]
    out_ref[...] = h + x


def _node_mlp(x, p0, p1, wa, wb, b1, w2, b2, gn, bbn):
    grid = (N // ROW_BLK_NODE,)
    row = lambda i: (i, 0)
    full = lambda i: (0, 0)
    return pl.pallas_call(
        _node_body,
        grid=grid,
        in_specs=[
            pl.BlockSpec((ROW_BLK_NODE, D), row),
            pl.BlockSpec((ROW_BLK_NODE, D), row),
            pl.BlockSpec((ROW_BLK_NODE, D), row),
            pl.BlockSpec((D, D), full),
            pl.BlockSpec((D, D), full),
            pl.BlockSpec((1, D), full),
            pl.BlockSpec((D, D), full),
            pl.BlockSpec((1, D), full),
            pl.BlockSpec((1, D), full),
            pl.BlockSpec((1, D), full),
        ],
        out_specs=pl.BlockSpec((ROW_BLK_NODE, D), row),
        out_shape=jax.ShapeDtypeStruct((N, D), jnp.float32),
    )(x, p0, p1, wa, wb, b1, w2, b2, gn, bbn)


def kernel(x, edge_attr, edge_index, shapes, we1, be1, we2, be2, ge, bbe,
           wn1, bn1, wn2, bn2, gn, bbn):
    del shapes
    src = edge_index[0]
    dst = edge_index[1]
    wi, wj, we = we1[:D], we1[D:2 * D], we1[2 * D:]
    px, qx = _precompute(x, wi, wj)
    gi, gj = _sc_gather(px, qx, dst, src)
    edges_new = _edge_mlp(gi, gj, edge_attr, we, be1[None, :], we2, be2[None, :],
                          ge[None, :], bbe[None, :])
    zs = jnp.zeros((NROWS_PER_TILE, D), jnp.float32)
    partials = _sc_scatter(edges_new, dst, zs)
    nodes_new = _node_mlp(x, partials[:N], partials[NPAD:NPAD + N], wn1[:D], wn1[D:],
                          bn1[None, :], wn2, bn2[None, :], gn[None, :],
                          bbn[None, :])
    return nodes_new, edges_new


# speedup vs baseline: 4.1664x; 1.4259x over previous
import functools

import jax
import jax.numpy as jnp
from jax import lax
from jax.experimental import pallas as pl
from jax.experimental.pallas import tpu as pltpu
from jax.experimental.pallas import tpu_sc as plsc

N = 10000
E = 320000
D = 128

SC_NC = 2
SC_NS = 16
SC_NW = SC_NC * SC_NS
E_PER_W = E // SC_NW
GCHUNK = 80
GSTEPS = E_PER_W // GCHUNK

ROW_BLK_PRE = 2000
ROW_BLK_EDGE = 1280
ROW_BLK_NODE = 2000


def _pre_body(x_ref, wi_ref, wj_ref, px_ref, qx_ref):
    x = x_ref[...]
    px_ref[...] = jnp.dot(x, wi_ref[...], preferred_element_type=jnp.float32)
    qx_ref[...] = jnp.dot(x, wj_ref[...], preferred_element_type=jnp.float32)


def _precompute(x, wi, wj):
    grid = (N // ROW_BLK_PRE,)
    return pl.pallas_call(
        _pre_body,
        grid=grid,
        in_specs=[
            pl.BlockSpec((ROW_BLK_PRE, D), lambda i: (i, 0)),
            pl.BlockSpec((D, D), lambda i: (0, 0)),
            pl.BlockSpec((D, D), lambda i: (0, 0)),
        ],
        out_specs=[
            pl.BlockSpec((ROW_BLK_PRE, D), lambda i: (i, 0)),
            pl.BlockSpec((ROW_BLK_PRE, D), lambda i: (i, 0)),
        ],
        out_shape=[
            jax.ShapeDtypeStruct((N, D), jnp.float32),
            jax.ShapeDtypeStruct((N, D), jnp.float32),
        ],
    )(x, wi, wj)


def _sc_gather_body(px_hbm, qx_hbm, dst_hbm, src_hbm, gi_hbm, gj_hbm,
                    di_v, si_v, ra0, ra1, rb0, rb1, sa0, sa1, sb0, sb1):
    wid = lax.axis_index("c") * SC_NS + lax.axis_index("s")
    wbase = wid * E_PER_W
    ras, rbs, sas, sbs = (ra0, ra1), (rb0, rb1), (sa0, sa1), (sb0, sb1)
    pltpu.sync_copy(dst_hbm.at[wid], di_v)
    pltpu.sync_copy(src_hbm.at[wid], si_v)

    def start(it, b):
        pltpu.async_copy(px_hbm.at[di_v.at[it]], ras[b], sas[b])
        pltpu.async_copy(qx_hbm.at[si_v.at[it]], rbs[b], sbs[b])

    def finish(it, b):
        pltpu.make_async_copy(px_hbm.at[di_v.at[it]], ras[b], sas[b]).wait()
        pltpu.make_async_copy(qx_hbm.at[si_v.at[it]], rbs[b], sbs[b]).wait()
        base = wbase + it * GCHUNK
        pltpu.sync_copy(ras[b], gi_hbm.at[pl.ds(base, GCHUNK)])
        pltpu.sync_copy(rbs[b], gj_hbm.at[pl.ds(base, GCHUNK)])

    start(0, 0)
    start(1, 1)

    def pair(k, _):
        it0 = k * 2
        finish(it0, 0)

        @pl.when(it0 + 2 < GSTEPS)
        def _():
            start(it0 + 2, 0)

        finish(it0 + 1, 1)

        @pl.when(it0 + 3 < GSTEPS)
        def _():
            start(it0 + 3, 1)

        return ()

    lax.fori_loop(0, GSTEPS // 2, pair, ())
    if GSTEPS % 2 == 1:
        finish(GSTEPS - 1, 0)


@functools.partial(
    pl.kernel,
    out_type=[
        jax.ShapeDtypeStruct((E, D), jnp.float32),
        jax.ShapeDtypeStruct((E, D), jnp.float32),
    ],
    mesh=plsc.VectorSubcoreMesh(core_axis_name="c", subcore_axis_name="s"),
    scratch_types=[
        pltpu.VMEM((GSTEPS, GCHUNK), jnp.int32),
        pltpu.VMEM((GSTEPS, GCHUNK), jnp.int32),
        pltpu.VMEM((GCHUNK, D), jnp.float32),
        pltpu.VMEM((GCHUNK, D), jnp.float32),
        pltpu.VMEM((GCHUNK, D), jnp.float32),
        pltpu.VMEM((GCHUNK, D), jnp.float32),
        pltpu.SemaphoreType.DMA,
        pltpu.SemaphoreType.DMA,
        pltpu.SemaphoreType.DMA,
        pltpu.SemaphoreType.DMA,
    ],
)
def _sc_gather(px_hbm, qx_hbm, dst_hbm, src_hbm, gi_hbm, gj_hbm,
               di_v, si_v, ra0, ra1, rb0, rb1, sa0, sa1, sb0, sb1):
    _sc_gather_body(px_hbm, qx_hbm, dst_hbm, src_hbm, gi_hbm, gj_hbm,
                    di_v, si_v, ra0, ra1, rb0, rb1, sa0, sa1, sb0, sb1)


NPAD = 10240
NROWS_PER_TILE = NPAD // SC_NS
SCHUNK = 80
SSTEPS = E_PER_W // SCHUNK


def _sc_scatter_body(edges_hbm, dst_hbm, zs_hbm, out_hbm, idx_v, r0, r1, acc_sh,
                     s0, s1):
    c = lax.axis_index("c")
    s = lax.axis_index("s")
    wid = c * SC_NS + s
    wbase = wid * E_PER_W
    rows, sems = (r0, r1), (s0, s1)
    pltpu.sync_copy(zs_hbm, acc_sh.at[pl.ds(s * NROWS_PER_TILE, NROWS_PER_TILE)])
    pltpu.sync_copy(dst_hbm.at[wid], idx_v)
    plsc.subcore_barrier()

    def start(it, b):
        base = wbase + it * SCHUNK
        pltpu.async_copy(edges_hbm.at[pl.ds(base, SCHUNK)], rows[b], sems[b])

    def finish(it, b):
        base = wbase + it * SCHUNK
        pltpu.make_async_copy(edges_hbm.at[pl.ds(base, SCHUNK)], rows[b],
                              sems[b]).wait()
        pltpu.sync_copy(rows[b], acc_sh.at[idx_v.at[it]], add=True)

    start(0, 0)
    start(1, 1)

    def pair(k, _):
        it0 = k * 2
        finish(it0, 0)

        @pl.when(it0 + 2 < SSTEPS)
        def _():
            start(it0 + 2, 0)

        finish(it0 + 1, 1)

        @pl.when(it0 + 3 < SSTEPS)
        def _():
            start(it0 + 3, 1)

        return ()

    lax.fori_loop(0, SSTEPS // 2, pair, ())
    if SSTEPS % 2 == 1:
        finish(SSTEPS - 1, 0)
    plsc.subcore_barrier()
    pltpu.sync_copy(acc_sh.at[pl.ds(s * NROWS_PER_TILE, NROWS_PER_TILE)],
                    out_hbm.at[pl.ds(c * NPAD + s * NROWS_PER_TILE,
                                     NROWS_PER_TILE)])


@functools.partial(
    pl.kernel,
    out_type=jax.ShapeDtypeStruct((2 * NPAD, D), jnp.float32),
    mesh=plsc.VectorSubcoreMesh(core_axis_name="c", subcore_axis_name="s"),
    scratch_types=[
        pltpu.VMEM((SSTEPS, SCHUNK), jnp.int32),
        pltpu.VMEM((SCHUNK, D), jnp.float32),
        pltpu.VMEM((SCHUNK, D), jnp.float32),
        pltpu.VMEM_SHARED((NPAD, D), jnp.float32),
        pltpu.SemaphoreType.DMA,
        pltpu.SemaphoreType.DMA,
    ],
)
def _sc_scatter(edges_hbm, dst_hbm, zs_hbm, out_hbm, idx_v, r0, r1, acc_sh,
                s0, s1):
    _sc_scatter_body(edges_hbm, dst_hbm, zs_hbm, out_hbm, idx_v, r0, r1, acc_sh,
                     s0, s1)


def _edge_body(gi_ref, gj_ref, ea_ref, we_ref, b1_ref, w2_ref, b2_ref, ge_ref,
               bbe_ref, out_ref):
    ea = ea_ref[...]
    h = gi_ref[...] + gj_ref[...] + jnp.dot(
        ea, we_ref[...], preferred_element_type=jnp.float32) + b1_ref[...]
    h = h * jax.nn.sigmoid(h)
    h = jnp.dot(h, w2_ref[...], preferred_element_type=jnp.float32) + b2_ref[...]
    mu = jnp.mean(h, axis=-1, keepdims=True)
    var = jnp.mean((h - mu) * (h - mu), axis=-1, keepdims=True)
    h = (h - mu) * jax.lax.rsqrt(var + 1e-5) * ge_ref[...] + bbe_ref[...]
    out_ref[...] = h + ea


def _edge_mlp(gi, gj, ea, we, b1, w2, b2, ge, bbe):
    grid = (E // ROW_BLK_EDGE,)
    row = lambda i: (i, 0)
    full = lambda i: (0, 0)
    return pl.pallas_call(
        _edge_body,
        grid=grid,
        in_specs=[
            pl.BlockSpec((ROW_BLK_EDGE, D), row),
            pl.BlockSpec((ROW_BLK_EDGE, D), row),
            pl.BlockSpec((ROW_BLK_EDGE, D), row),
            pl.BlockSpec((D, D), full),
            pl.BlockSpec((1, D), full),
            pl.BlockSpec((D, D), full),
            pl.BlockSpec((1, D), full),
            pl.BlockSpec((1, D), full),
            pl.BlockSpec((1, D), full),
        ],
        out_specs=pl.BlockSpec((ROW_BLK_EDGE, D), row),
        out_shape=jax.ShapeDtypeStruct((E, D), jnp.float32),
    )(gi, gj, ea, we, b1, w2, b2, ge, bbe)


def _node_body(x_ref, p0_ref, p1_ref, wa_ref, wb_ref, b1_ref, w2_ref, b2_ref,
               gn_ref, bbn_ref, out_ref):
    x = x_ref[...]
    agg = p0_ref[...] + p1_ref[...]
    h = (jnp.dot(x, wa_ref[...], preferred_element_type=jnp.float32)
         + jnp.dot(agg, wb_ref[...], preferred_element_type=jnp.float32)
         + b1_ref[...])
    h = h * jax.nn.sigmoid(h)
    h = jnp.dot(h, w2_ref[...], preferred_element_type=jnp.float32) + b2_ref[...]
    mu = jnp.mean(h, axis=-1, keepdims=True)
    var = jnp.mean((h - mu) * (h - mu), axis=-1, keepdims=True)
    h = (h - mu) * jax.lax.rsqrt(var + 1e-5) * gn_ref[...] + bbn_ref[...]
    out_ref[...] = h + x


def _node_mlp(x, p0, p1, wa, wb, b1, w2, b2, gn, bbn):
    grid = (N // ROW_BLK_NODE,)
    row = lambda i: (i, 0)
    full = lambda i: (0, 0)
    return pl.pallas_call(
        _node_body,
        grid=grid,
        in_specs=[
            pl.BlockSpec((ROW_BLK_NODE, D), row),
            pl.BlockSpec((ROW_BLK_NODE, D), row),
            pl.BlockSpec((ROW_BLK_NODE, D), row),
            pl.BlockSpec((D, D), full),
            pl.BlockSpec((D, D), full),
            pl.BlockSpec((1, D), full),
            pl.BlockSpec((D, D), full),
            pl.BlockSpec((1, D), full),
            pl.BlockSpec((1, D), full),
            pl.BlockSpec((1, D), full),
        ],
        out_specs=pl.BlockSpec((ROW_BLK_NODE, D), row),
        out_shape=jax.ShapeDtypeStruct((N, D), jnp.float32),
    )(x, p0, p1, wa, wb, b1, w2, b2, gn, bbn)


def kernel(x, edge_attr, edge_index, shapes, we1, be1, we2, be2, ge, bbe,
           wn1, bn1, wn2, bn2, gn, bbn):
    del shapes
    src = edge_index[0]
    dst = edge_index[1]
    wi, wj, we = we1[:D], we1[D:2 * D], we1[2 * D:]
    px, qx = _precompute(x, wi, wj)
    dst3 = dst.reshape(SC_NW, GSTEPS, GCHUNK)
    src3 = src.reshape(SC_NW, GSTEPS, GCHUNK)
    gi, gj = _sc_gather(px, qx, dst3, src3)
    edges_new = _edge_mlp(gi, gj, edge_attr, we, be1[None, :], we2, be2[None, :],
                          ge[None, :], bbe[None, :])
    zs = jnp.zeros((NROWS_PER_TILE, D), jnp.float32)
    partials = _sc_scatter(edges_new, dst3, zs)
    nodes_new = _node_mlp(x, partials[:N], partials[NPAD:NPAD + N], wn1[:D], wn1[D:],
                          bn1[None, :], wn2, bn2[None, :], gn[None, :],
                          bbn[None, :])
    return nodes_new, edges_new


# speedup vs baseline: 4.7497x; 1.1400x over previous
import functools

import jax
import jax.numpy as jnp
from jax import lax
from jax.experimental import pallas as pl
from jax.experimental.pallas import tpu as pltpu
from jax.experimental.pallas import tpu_sc as plsc

N = 10000
E = 320000
D = 128

SC_NC = 2
SC_NS = 16
SC_NW = SC_NC * SC_NS
E_PER_W = E // SC_NW
GCHUNK = 80
GSTEPS = E_PER_W // GCHUNK

ROW_BLK_PRE = 2000
ROW_BLK_EDGE = 2560
ROW_BLK_NODE = 2000


def _pre_body(x_ref, wi_ref, wj_ref, px_ref, qx_ref):
    x = x_ref[...]
    px_ref[...] = jnp.dot(x, wi_ref[...], preferred_element_type=jnp.float32)
    qx_ref[...] = jnp.dot(x, wj_ref[...], preferred_element_type=jnp.float32)


def _precompute(x, wi, wj):
    grid = (N // ROW_BLK_PRE,)
    return pl.pallas_call(
        _pre_body,
        grid=grid,
        in_specs=[
            pl.BlockSpec((ROW_BLK_PRE, D), lambda i: (i, 0)),
            pl.BlockSpec((D, D), lambda i: (0, 0)),
            pl.BlockSpec((D, D), lambda i: (0, 0)),
        ],
        out_specs=[
            pl.BlockSpec((ROW_BLK_PRE, D), lambda i: (i, 0)),
            pl.BlockSpec((ROW_BLK_PRE, D), lambda i: (i, 0)),
        ],
        out_shape=[
            jax.ShapeDtypeStruct((N, D), jnp.float32),
            jax.ShapeDtypeStruct((N, D), jnp.float32),
        ],
    )(x, wi, wj)


def _sc_gather_body(px_hbm, qx_hbm, dst_hbm, src_hbm, gi_hbm, gj_hbm,
                    di_v, si_v, ra0, ra1, rb0, rb1, sa0, sa1, sb0, sb1):
    wid = lax.axis_index("c") * SC_NS + lax.axis_index("s")
    wbase = wid * E_PER_W
    ras, rbs, sas, sbs = (ra0, ra1), (rb0, rb1), (sa0, sa1), (sb0, sb1)
    pltpu.sync_copy(dst_hbm.at[wid], di_v)
    pltpu.sync_copy(src_hbm.at[wid], si_v)

    def start(it, b):
        pltpu.async_copy(px_hbm.at[di_v.at[it]], ras[b], sas[b])
        pltpu.async_copy(qx_hbm.at[si_v.at[it]], rbs[b], sbs[b])

    def finish(it, b):
        pltpu.make_async_copy(px_hbm.at[di_v.at[it]], ras[b], sas[b]).wait()
        pltpu.make_async_copy(qx_hbm.at[si_v.at[it]], rbs[b], sbs[b]).wait()
        base = wbase + it * GCHUNK
        pltpu.sync_copy(ras[b], gi_hbm.at[pl.ds(base, GCHUNK)])
        pltpu.sync_copy(rbs[b], gj_hbm.at[pl.ds(base, GCHUNK)])

    start(0, 0)
    start(1, 1)

    def pair(k, _):
        it0 = k * 2
        finish(it0, 0)

        @pl.when(it0 + 2 < GSTEPS)
        def _():
            start(it0 + 2, 0)

        finish(it0 + 1, 1)

        @pl.when(it0 + 3 < GSTEPS)
        def _():
            start(it0 + 3, 1)

        return ()

    lax.fori_loop(0, GSTEPS // 2, pair, ())
    if GSTEPS % 2 == 1:
        finish(GSTEPS - 1, 0)


@functools.partial(
    pl.kernel,
    out_type=[
        jax.ShapeDtypeStruct((E, D), jnp.float32),
        jax.ShapeDtypeStruct((E, D), jnp.float32),
    ],
    mesh=plsc.VectorSubcoreMesh(core_axis_name="c", subcore_axis_name="s"),
    scratch_types=[
        pltpu.VMEM((GSTEPS, GCHUNK), jnp.int32),
        pltpu.VMEM((GSTEPS, GCHUNK), jnp.int32),
        pltpu.VMEM((GCHUNK, D), jnp.float32),
        pltpu.VMEM((GCHUNK, D), jnp.float32),
        pltpu.VMEM((GCHUNK, D), jnp.float32),
        pltpu.VMEM((GCHUNK, D), jnp.float32),
        pltpu.SemaphoreType.DMA,
        pltpu.SemaphoreType.DMA,
        pltpu.SemaphoreType.DMA,
        pltpu.SemaphoreType.DMA,
    ],
)
def _sc_gather(px_hbm, qx_hbm, dst_hbm, src_hbm, gi_hbm, gj_hbm,
               di_v, si_v, ra0, ra1, rb0, rb1, sa0, sa1, sb0, sb1):
    _sc_gather_body(px_hbm, qx_hbm, dst_hbm, src_hbm, gi_hbm, gj_hbm,
                    di_v, si_v, ra0, ra1, rb0, rb1, sa0, sa1, sb0, sb1)


NPAD = 10240
NROWS_PER_TILE = NPAD // SC_NS
SCHUNK = 80
SSTEPS = E_PER_W // SCHUNK


def _sc_scatter_body(edges_hbm, dst_hbm, zs_hbm, out_hbm, idx_v, r0, r1, acc_sh,
                     s0, s1):
    c = lax.axis_index("c")
    s = lax.axis_index("s")
    wid = c * SC_NS + s
    wbase = wid * E_PER_W
    rows, sems = (r0, r1), (s0, s1)
    pltpu.sync_copy(zs_hbm, acc_sh.at[pl.ds(s * NROWS_PER_TILE, NROWS_PER_TILE)])
    pltpu.sync_copy(dst_hbm.at[wid], idx_v)
    plsc.subcore_barrier()

    def start(it, b):
        base = wbase + it * SCHUNK
        pltpu.async_copy(edges_hbm.at[pl.ds(base, SCHUNK)], rows[b], sems[b])

    def finish(it, b):
        base = wbase + it * SCHUNK
        pltpu.make_async_copy(edges_hbm.at[pl.ds(base, SCHUNK)], rows[b],
                              sems[b]).wait()
        pltpu.sync_copy(rows[b], acc_sh.at[idx_v.at[it]], add=True)

    start(0, 0)
    start(1, 1)

    def pair(k, _):
        it0 = k * 2
        finish(it0, 0)

        @pl.when(it0 + 2 < SSTEPS)
        def _():
            start(it0 + 2, 0)

        finish(it0 + 1, 1)

        @pl.when(it0 + 3 < SSTEPS)
        def _():
            start(it0 + 3, 1)

        return ()

    lax.fori_loop(0, SSTEPS // 2, pair, ())
    if SSTEPS % 2 == 1:
        finish(SSTEPS - 1, 0)
    plsc.subcore_barrier()
    pltpu.sync_copy(acc_sh.at[pl.ds(s * NROWS_PER_TILE, NROWS_PER_TILE)],
                    out_hbm.at[pl.ds(c * NPAD + s * NROWS_PER_TILE,
                                     NROWS_PER_TILE)])


@functools.partial(
    pl.kernel,
    out_type=jax.ShapeDtypeStruct((2 * NPAD, D), jnp.float32),
    mesh=plsc.VectorSubcoreMesh(core_axis_name="c", subcore_axis_name="s"),
    scratch_types=[
        pltpu.VMEM((SSTEPS, SCHUNK), jnp.int32),
        pltpu.VMEM((SCHUNK, D), jnp.float32),
        pltpu.VMEM((SCHUNK, D), jnp.float32),
        pltpu.VMEM_SHARED((NPAD, D), jnp.float32),
        pltpu.SemaphoreType.DMA,
        pltpu.SemaphoreType.DMA,
    ],
)
def _sc_scatter(edges_hbm, dst_hbm, zs_hbm, out_hbm, idx_v, r0, r1, acc_sh,
                s0, s1):
    _sc_scatter_body(edges_hbm, dst_hbm, zs_hbm, out_hbm, idx_v, r0, r1, acc_sh,
                     s0, s1)


def _edge_body(gi_ref, gj_ref, ea_ref, we_ref, b1_ref, w2_ref, b2_ref, ge_ref,
               bbe_ref, out_ref):
    ea = ea_ref[...]
    h = (gi_ref[...] + gj_ref[...]
         + jnp.dot(ea, we_ref[...], preferred_element_type=jnp.float32)
         + b1_ref[...])
    h = h * jax.nn.sigmoid(h)
    h = jnp.dot(h, w2_ref[...], preferred_element_type=jnp.float32) + b2_ref[...]
    mu = jnp.mean(h, axis=-1, keepdims=True)
    var = jnp.mean((h - mu) * (h - mu), axis=-1, keepdims=True)
    h = (h - mu) * jax.lax.rsqrt(var + 1e-5) * ge_ref[...] + bbe_ref[...]
    out_ref[...] = h + ea


def _edge_mlp(gi, gj, ea, we, b1, w2, b2, ge, bbe):
    grid = (E // ROW_BLK_EDGE,)
    row = lambda i: (i, 0)
    full = lambda i: (0, 0)
    return pl.pallas_call(
        _edge_body,
        grid=grid,
        in_specs=[
            pl.BlockSpec((ROW_BLK_EDGE, D), row),
            pl.BlockSpec((ROW_BLK_EDGE, D), row),
            pl.BlockSpec((ROW_BLK_EDGE, D), row),
            pl.BlockSpec((D, D), full),
            pl.BlockSpec((1, D), full),
            pl.BlockSpec((D, D), full),
            pl.BlockSpec((1, D), full),
            pl.BlockSpec((1, D), full),
            pl.BlockSpec((1, D), full),
        ],
        out_specs=pl.BlockSpec((ROW_BLK_EDGE, D), row),
        out_shape=jax.ShapeDtypeStruct((E, D), jnp.float32),
    )(gi, gj, ea, we, b1, w2, b2, ge, bbe)


def _node_body(x_ref, p0_ref, p1_ref, wa_ref, wb_ref, b1_ref, w2_ref, b2_ref,
               gn_ref, bbn_ref, out_ref):
    x = x_ref[...]
    agg = p0_ref[...] + p1_ref[...]
    h = (jnp.dot(x, wa_ref[...], preferred_element_type=jnp.float32)
         + jnp.dot(agg, wb_ref[...], preferred_element_type=jnp.float32)
         + b1_ref[...])
    h = h * jax.nn.sigmoid(h)
    h = jnp.dot(h, w2_ref[...], preferred_element_type=jnp.float32) + b2_ref[...]
    mu = jnp.mean(h, axis=-1, keepdims=True)
    var = jnp.mean((h - mu) * (h - mu), axis=-1, keepdims=True)
    h = (h - mu) * jax.lax.rsqrt(var + 1e-5) * gn_ref[...] + bbn_ref[...]
    out_ref[...] = h + x


def _node_mlp(x, p0, p1, wa, wb, b1, w2, b2, gn, bbn):
    grid = (N // ROW_BLK_NODE,)
    row = lambda i: (i, 0)
    full = lambda i: (0, 0)
    return pl.pallas_call(
        _node_body,
        grid=grid,
        in_specs=[
            pl.BlockSpec((ROW_BLK_NODE, D), row),
            pl.BlockSpec((ROW_BLK_NODE, D), row),
            pl.BlockSpec((ROW_BLK_NODE, D), row),
            pl.BlockSpec((D, D), full),
            pl.BlockSpec((D, D), full),
            pl.BlockSpec((1, D), full),
            pl.BlockSpec((D, D), full),
            pl.BlockSpec((1, D), full),
            pl.BlockSpec((1, D), full),
            pl.BlockSpec((1, D), full),
        ],
        out_specs=pl.BlockSpec((ROW_BLK_NODE, D), row),
        out_shape=jax.ShapeDtypeStruct((N, D), jnp.float32),
    )(x, p0, p1, wa, wb, b1, w2, b2, gn, bbn)


def kernel(x, edge_attr, edge_index, shapes, we1, be1, we2, be2, ge, bbe,
           wn1, bn1, wn2, bn2, gn, bbn):
    del shapes
    src = edge_index[0]
    dst = edge_index[1]
    wi, wj, we = we1[:D], we1[D:2 * D], we1[2 * D:]
    px, qx = _precompute(x, wi, wj)
    dst3 = dst.reshape(SC_NW, GSTEPS, GCHUNK)
    src3 = src.reshape(SC_NW, GSTEPS, GCHUNK)
    gi, gj = _sc_gather(px, qx, dst3, src3)
    edges_new = _edge_mlp(gi, gj, edge_attr, we, be1[None, :], we2, be2[None, :],
                          ge[None, :], bbe[None, :])
    zs = jnp.zeros((NROWS_PER_TILE, D), jnp.float32)
    partials = _sc_scatter(edges_new, dst3, zs)
    nodes_new = _node_mlp(x, partials[:N], partials[NPAD:NPAD + N], wn1[:D],
                          wn1[D:], bn1[None, :], wn2, bn2[None, :], gn[None, :],
                          bbn[None, :])
    return nodes_new, edges_new


# speedup vs baseline: 5.3143x; 1.1189x over previous
import functools

import jax
import jax.numpy as jnp
from jax import lax
from jax.experimental import pallas as pl
from jax.experimental.pallas import tpu as pltpu
from jax.experimental.pallas import tpu_sc as plsc

N = 10000
E = 320000
D = 128

SC_NC = 2
SC_NS = 16
SC_NW = SC_NC * SC_NS
E_PER_W = E // SC_NW
GCHUNK = 80
GSTEPS = E_PER_W // GCHUNK

ROW_BLK_PRE = 2000
ROW_BLK_EDGE = 2560
ROW_BLK_NODE = 2000


def _pre_body(x_ref, wi_ref, wj_ref, px_ref, qx_ref):
    x = x_ref[...]
    px_ref[...] = jnp.dot(x, wi_ref[...], preferred_element_type=jnp.float32)
    qx_ref[...] = jnp.dot(x, wj_ref[...], preferred_element_type=jnp.float32)


def _precompute(x, wi, wj):
    grid = (N // ROW_BLK_PRE,)
    return pl.pallas_call(
        _pre_body,
        grid=grid,
        in_specs=[
            pl.BlockSpec((ROW_BLK_PRE, D), lambda i: (i, 0)),
            pl.BlockSpec((D, D), lambda i: (0, 0)),
            pl.BlockSpec((D, D), lambda i: (0, 0)),
        ],
        out_specs=[
            pl.BlockSpec((ROW_BLK_PRE, D), lambda i: (i, 0)),
            pl.BlockSpec((ROW_BLK_PRE, D), lambda i: (i, 0)),
        ],
        out_shape=[
            jax.ShapeDtypeStruct((N, D), jnp.float32),
            jax.ShapeDtypeStruct((N, D), jnp.float32),
        ],
    )(x, wi, wj)


def _sc_gather_body(px_hbm, qx_hbm, dst_hbm, src_hbm, g_hbm,
                    di_v, si_v, ra0, ra1, rb0, rb1, sa0, sa1, sb0, sb1):
    wid = lax.axis_index("c") * SC_NS + lax.axis_index("s")
    wbase = wid * E_PER_W
    ras, rbs, sas, sbs = (ra0, ra1), (rb0, rb1), (sa0, sa1), (sb0, sb1)
    pltpu.sync_copy(dst_hbm.at[wid], di_v)
    pltpu.sync_copy(src_hbm.at[wid], si_v)

    def start(it, b):
        pltpu.async_copy(px_hbm.at[di_v.at[it]], ras[b], sas[b])
        pltpu.async_copy(qx_hbm.at[si_v.at[it]], rbs[b], sbs[b])

    def finish(it, b):
        pltpu.make_async_copy(px_hbm.at[di_v.at[it]], ras[b], sas[b]).wait()
        pltpu.make_async_copy(qx_hbm.at[si_v.at[it]], rbs[b], sbs[b]).wait()
        ra, rb = ras[b], rbs[b]

        def add_row(r, _):
            for k in range(D // 16):
                sl = pl.ds(k * 16, 16)
                ra[r, sl] = ra[r, sl] + rb[r, sl]
            return ()

        lax.fori_loop(0, GCHUNK, add_row, ())
        base = wbase + it * GCHUNK
        pltpu.sync_copy(ra, g_hbm.at[pl.ds(base, GCHUNK)])

    start(0, 0)
    start(1, 1)

    def pair(k, _):
        it0 = k * 2
        finish(it0, 0)

        @pl.when(it0 + 2 < GSTEPS)
        def _():
            start(it0 + 2, 0)

        finish(it0 + 1, 1)

        @pl.when(it0 + 3 < GSTEPS)
        def _():
            start(it0 + 3, 1)

        return ()

    lax.fori_loop(0, GSTEPS // 2, pair, ())
    if GSTEPS % 2 == 1:
        finish(GSTEPS - 1, 0)


@functools.partial(
    pl.kernel,
    out_type=jax.ShapeDtypeStruct((E, D), jnp.float32),
    mesh=plsc.VectorSubcoreMesh(core_axis_name="c", subcore_axis_name="s"),
    scratch_types=[
        pltpu.VMEM((GSTEPS, GCHUNK), jnp.int32),
        pltpu.VMEM((GSTEPS, GCHUNK), jnp.int32),
        pltpu.VMEM((GCHUNK, D), jnp.float32),
        pltpu.VMEM((GCHUNK, D), jnp.float32),
        pltpu.VMEM((GCHUNK, D), jnp.float32),
        pltpu.VMEM((GCHUNK, D), jnp.float32),
        pltpu.SemaphoreType.DMA,
        pltpu.SemaphoreType.DMA,
        pltpu.SemaphoreType.DMA,
        pltpu.SemaphoreType.DMA,
    ],
)
def _sc_gather(px_hbm, qx_hbm, dst_hbm, src_hbm, g_hbm,
               di_v, si_v, ra0, ra1, rb0, rb1, sa0, sa1, sb0, sb1):
    _sc_gather_body(px_hbm, qx_hbm, dst_hbm, src_hbm, g_hbm,
                    di_v, si_v, ra0, ra1, rb0, rb1, sa0, sa1, sb0, sb1)


NPAD = 10240
NROWS_PER_TILE = NPAD // SC_NS
SCHUNK = 80
SSTEPS = E_PER_W // SCHUNK


def _sc_scatter_body(edges_hbm, dst_hbm, zs_hbm, out_hbm, idx_v, r0, r1, acc_sh,
                     s0, s1):
    c = lax.axis_index("c")
    s = lax.axis_index("s")
    wid = c * SC_NS + s
    wbase = wid * E_PER_W
    rows, sems = (r0, r1), (s0, s1)
    pltpu.sync_copy(zs_hbm, acc_sh.at[pl.ds(s * NROWS_PER_TILE, NROWS_PER_TILE)])
    pltpu.sync_copy(dst_hbm.at[wid], idx_v)
    plsc.subcore_barrier()

    def start(it, b):
        base = wbase + it * SCHUNK
        pltpu.async_copy(edges_hbm.at[pl.ds(base, SCHUNK)], rows[b], sems[b])

    def finish(it, b):
        base = wbase + it * SCHUNK
        pltpu.make_async_copy(edges_hbm.at[pl.ds(base, SCHUNK)], rows[b],
                              sems[b]).wait()
        pltpu.sync_copy(rows[b], acc_sh.at[idx_v.at[it]], add=True)

    start(0, 0)
    start(1, 1)

    def pair(k, _):
        it0 = k * 2
        finish(it0, 0)

        @pl.when(it0 + 2 < SSTEPS)
        def _():
            start(it0 + 2, 0)

        finish(it0 + 1, 1)

        @pl.when(it0 + 3 < SSTEPS)
        def _():
            start(it0 + 3, 1)

        return ()

    lax.fori_loop(0, SSTEPS // 2, pair, ())
    if SSTEPS % 2 == 1:
        finish(SSTEPS - 1, 0)
    plsc.subcore_barrier()
    pltpu.sync_copy(acc_sh.at[pl.ds(s * NROWS_PER_TILE, NROWS_PER_TILE)],
                    out_hbm.at[pl.ds(c * NPAD + s * NROWS_PER_TILE,
                                     NROWS_PER_TILE)])


@functools.partial(
    pl.kernel,
    out_type=jax.ShapeDtypeStruct((2 * NPAD, D), jnp.float32),
    mesh=plsc.VectorSubcoreMesh(core_axis_name="c", subcore_axis_name="s"),
    scratch_types=[
        pltpu.VMEM((SSTEPS, SCHUNK), jnp.int32),
        pltpu.VMEM((SCHUNK, D), jnp.float32),
        pltpu.VMEM((SCHUNK, D), jnp.float32),
        pltpu.VMEM_SHARED((NPAD, D), jnp.float32),
        pltpu.SemaphoreType.DMA,
        pltpu.SemaphoreType.DMA,
    ],
)
def _sc_scatter(edges_hbm, dst_hbm, zs_hbm, out_hbm, idx_v, r0, r1, acc_sh,
                s0, s1):
    _sc_scatter_body(edges_hbm, dst_hbm, zs_hbm, out_hbm, idx_v, r0, r1, acc_sh,
                     s0, s1)


def _edge_body(g_ref, ea_ref, we_ref, b1_ref, w2_ref, b2_ref, ge_ref,
               bbe_ref, out_ref):
    ea = ea_ref[...]
    h = (g_ref[...]
         + jnp.dot(ea, we_ref[...], preferred_element_type=jnp.float32)
         + b1_ref[...])
    h = h * jax.nn.sigmoid(h)
    h = jnp.dot(h, w2_ref[...], preferred_element_type=jnp.float32) + b2_ref[...]
    mu = jnp.mean(h, axis=-1, keepdims=True)
    var = jnp.mean((h - mu) * (h - mu), axis=-1, keepdims=True)
    h = (h - mu) * jax.lax.rsqrt(var + 1e-5) * ge_ref[...] + bbe_ref[...]
    out_ref[...] = h + ea


def _edge_mlp(g, ea, we, b1, w2, b2, ge, bbe):
    grid = (E // ROW_BLK_EDGE,)
    row = lambda i: (i, 0)
    full = lambda i: (0, 0)
    return pl.pallas_call(
        _edge_body,
        grid=grid,
        in_specs=[
            pl.BlockSpec((ROW_BLK_EDGE, D), row),
            pl.BlockSpec((ROW_BLK_EDGE, D), row),
            pl.BlockSpec((D, D), full),
            pl.BlockSpec((1, D), full),
            pl.BlockSpec((D, D), full),
            pl.BlockSpec((1, D), full),
            pl.BlockSpec((1, D), full),
            pl.BlockSpec((1, D), full),
        ],
        out_specs=pl.BlockSpec((ROW_BLK_EDGE, D), row),
        out_shape=jax.ShapeDtypeStruct((E, D), jnp.float32),
    )(g, ea, we, b1, w2, b2, ge, bbe)


def _node_body(x_ref, p0_ref, p1_ref, wa_ref, wb_ref, b1_ref, w2_ref, b2_ref,
               gn_ref, bbn_ref, out_ref):
    x = x_ref[...]
    agg = p0_ref[...] + p1_ref[...]
    h = (jnp.dot(x, wa_ref[...], preferred_element_type=jnp.float32)
         + jnp.dot(agg, wb_ref[...], preferred_element_type=jnp.float32)
         + b1_ref[...])
    h = h * jax.nn.sigmoid(h)
    h = jnp.dot(h, w2_ref[...], preferred_element_type=jnp.float32) + b2_ref[...]
    mu = jnp.mean(h, axis=-1, keepdims=True)
    var = jnp.mean((h - mu) * (h - mu), axis=-1, keepdims=True)
    h = (h - mu) * jax.lax.rsqrt(var + 1e-5) * gn_ref[...] + bbn_ref[...]
    out_ref[...] = h + x


def _node_mlp(x, p0, p1, wa, wb, b1, w2, b2, gn, bbn):
    grid = (N // ROW_BLK_NODE,)
    row = lambda i: (i, 0)
    full = lambda i: (0, 0)
    return pl.pallas_call(
        _node_body,
        grid=grid,
        in_specs=[
            pl.BlockSpec((ROW_BLK_NODE, D), row),
            pl.BlockSpec((ROW_BLK_NODE, D), row),
            pl.BlockSpec((ROW_BLK_NODE, D), row),
            pl.BlockSpec((D, D), full),
            pl.BlockSpec((D, D), full),
            pl.BlockSpec((1, D), full),
            pl.BlockSpec((D, D), full),
            pl.BlockSpec((1, D), full),
            pl.BlockSpec((1, D), full),
            pl.BlockSpec((1, D), full),
        ],
        out_specs=pl.BlockSpec((ROW_BLK_NODE, D), row),
        out_shape=jax.ShapeDtypeStruct((N, D), jnp.float32),
    )(x, p0, p1, wa, wb, b1, w2, b2, gn, bbn)


def kernel(x, edge_attr, edge_index, shapes, we1, be1, we2, be2, ge, bbe,
           wn1, bn1, wn2, bn2, gn, bbn):
    del shapes
    src = edge_index[0]
    dst = edge_index[1]
    wi, wj, we = we1[:D], we1[D:2 * D], we1[2 * D:]
    px, qx = _precompute(x, wi, wj)
    dst3 = dst.reshape(SC_NW, GSTEPS, GCHUNK)
    src3 = src.reshape(SC_NW, GSTEPS, GCHUNK)
    g = _sc_gather(px, qx, dst3, src3)
    edges_new = _edge_mlp(g, edge_attr, we, be1[None, :], we2, be2[None, :],
                          ge[None, :], bbe[None, :])
    zs = jnp.zeros((NROWS_PER_TILE, D), jnp.float32)
    partials = _sc_scatter(edges_new, dst3, zs)
    nodes_new = _node_mlp(x, partials[:N], partials[NPAD:NPAD + N], wn1[:D],
                          wn1[D:], bn1[None, :], wn2, bn2[None, :], gn[None, :],
                          bbn[None, :])
    return nodes_new, edges_new


# speedup vs baseline: 5.8747x; 1.1054x over previous
import functools

import jax
import jax.numpy as jnp
from jax import lax
from jax.experimental import pallas as pl
from jax.experimental.pallas import tpu as pltpu
from jax.experimental.pallas import tpu_sc as plsc

N = 10000
E = 320000
D = 128

SC_NC = 2
SC_NS = 16
SC_NW = SC_NC * SC_NS
E_PER_W = E // SC_NW
GCHUNK = 80
GSTEPS = E_PER_W // GCHUNK

ROW_BLK_PRE = 2000
ROW_BLK_EDGE = 6400
ROW_BLK_NODE = 2000


def _pre_body(x_ref, wi_ref, wj_ref, px_ref, qx_ref):
    x = x_ref[...]
    px_ref[...] = jnp.dot(x, wi_ref[...], preferred_element_type=jnp.float32)
    qx_ref[...] = jnp.dot(x, wj_ref[...], preferred_element_type=jnp.float32)


def _precompute(x, wi, wj):
    grid = (N // ROW_BLK_PRE,)
    return pl.pallas_call(
        _pre_body,
        grid=grid,
        in_specs=[
            pl.BlockSpec((ROW_BLK_PRE, D), lambda i: (i, 0)),
            pl.BlockSpec((D, D), lambda i: (0, 0)),
            pl.BlockSpec((D, D), lambda i: (0, 0)),
        ],
        out_specs=[
            pl.BlockSpec((ROW_BLK_PRE, D), lambda i: (i, 0)),
            pl.BlockSpec((ROW_BLK_PRE, D), lambda i: (i, 0)),
        ],
        out_shape=[
            jax.ShapeDtypeStruct((N, D), jnp.float32),
            jax.ShapeDtypeStruct((N, D), jnp.float32),
        ],
    )(x, wi, wj)


def _sc_gather_body(px_hbm, qx_hbm, dst_hbm, src_hbm, g_hbm,
                    di_v, si_v, ra0, ra1, rb0, rb1, rc0, rc1,
                    sa0, sa1, sb0, sb1, sc0, sc1):
    wid = lax.axis_index("c") * SC_NS + lax.axis_index("s")
    wbase = wid * E_PER_W
    ras, rbs, rcs = (ra0, ra1), (rb0, rb1), (rc0, rc1)
    sas, sbs, scs = (sa0, sa1), (sb0, sb1), (sc0, sc1)
    pltpu.sync_copy(dst_hbm.at[wid], di_v)
    pltpu.sync_copy(src_hbm.at[wid], si_v)

    def start(it, b):
        pltpu.async_copy(px_hbm.at[di_v.at[it]], ras[b], sas[b])
        pltpu.async_copy(qx_hbm.at[si_v.at[it]], rbs[b], sbs[b])

    def wb_wait(it, b):
        base = wbase + it * GCHUNK
        pltpu.make_async_copy(rcs[b], g_hbm.at[pl.ds(base, GCHUNK)],
                              scs[b]).wait()

    def finish(it, b):
        pltpu.make_async_copy(px_hbm.at[di_v.at[it]], ras[b], sas[b]).wait()
        pltpu.make_async_copy(qx_hbm.at[si_v.at[it]], rbs[b], sbs[b]).wait()

        @pl.when(it >= 2)
        def _():
            wb_wait(it - 2, b)

        ra, rb, rc = ras[b], rbs[b], rcs[b]

        def add_row(r, _):
            for k in range(D // 16):
                sl = pl.ds(k * 16, 16)
                rc[r, sl] = ra[r, sl] + rb[r, sl]
            return ()

        lax.fori_loop(0, GCHUNK, add_row, ())
        base = wbase + it * GCHUNK
        pltpu.async_copy(rc, g_hbm.at[pl.ds(base, GCHUNK)], scs[b])

    start(0, 0)
    start(1, 1)

    def pair(k, _):
        it0 = k * 2
        finish(it0, 0)

        @pl.when(it0 + 2 < GSTEPS)
        def _():
            start(it0 + 2, 0)

        finish(it0 + 1, 1)

        @pl.when(it0 + 3 < GSTEPS)
        def _():
            start(it0 + 3, 1)

        return ()

    lax.fori_loop(0, GSTEPS // 2, pair, ())
    if GSTEPS % 2 == 1:
        finish(GSTEPS - 1, 0)
        wb_wait(GSTEPS - 2, 1)
        wb_wait(GSTEPS - 1, 0)
    else:
        wb_wait(GSTEPS - 2, 0)
        wb_wait(GSTEPS - 1, 1)


@functools.partial(
    pl.kernel,
    out_type=jax.ShapeDtypeStruct((E, D), jnp.float32),
    mesh=plsc.VectorSubcoreMesh(core_axis_name="c", subcore_axis_name="s"),
    scratch_types=[
        pltpu.VMEM((GSTEPS, GCHUNK), jnp.int32),
        pltpu.VMEM((GSTEPS, GCHUNK), jnp.int32),
        pltpu.VMEM((GCHUNK, D), jnp.float32),
        pltpu.VMEM((GCHUNK, D), jnp.float32),
        pltpu.VMEM((GCHUNK, D), jnp.float32),
        pltpu.VMEM((GCHUNK, D), jnp.float32),
        pltpu.VMEM((GCHUNK, D), jnp.float32),
        pltpu.VMEM((GCHUNK, D), jnp.float32),
        pltpu.SemaphoreType.DMA,
        pltpu.SemaphoreType.DMA,
        pltpu.SemaphoreType.DMA,
        pltpu.SemaphoreType.DMA,
        pltpu.SemaphoreType.DMA,
        pltpu.SemaphoreType.DMA,
    ],
)
def _sc_gather(px_hbm, qx_hbm, dst_hbm, src_hbm, g_hbm,
               di_v, si_v, ra0, ra1, rb0, rb1, rc0, rc1,
               sa0, sa1, sb0, sb1, sc0, sc1):
    _sc_gather_body(px_hbm, qx_hbm, dst_hbm, src_hbm, g_hbm,
                    di_v, si_v, ra0, ra1, rb0, rb1, rc0, rc1,
                    sa0, sa1, sb0, sb1, sc0, sc1)


NPAD = 10240
NROWS_PER_TILE = NPAD // SC_NS
SCHUNK = 80
SSTEPS = E_PER_W // SCHUNK


def _sc_scatter_body(edges_hbm, dst_hbm, zs_hbm, out_hbm, idx_v, r0, r1, acc_sh,
                     s0, s1):
    c = lax.axis_index("c")
    s = lax.axis_index("s")
    wid = c * SC_NS + s
    wbase = wid * E_PER_W
    rows, sems = (r0, r1), (s0, s1)
    pltpu.sync_copy(zs_hbm, acc_sh.at[pl.ds(s * NROWS_PER_TILE, NROWS_PER_TILE)])
    pltpu.sync_copy(dst_hbm.at[wid], idx_v)
    plsc.subcore_barrier()

    def start(it, b):
        base = wbase + it * SCHUNK
        pltpu.async_copy(edges_hbm.at[pl.ds(base, SCHUNK)], rows[b], sems[b])

    def finish(it, b):
        base = wbase + it * SCHUNK
        pltpu.make_async_copy(edges_hbm.at[pl.ds(base, SCHUNK)], rows[b],
                              sems[b]).wait()
        pltpu.sync_copy(rows[b], acc_sh.at[idx_v.at[it]], add=True)

    start(0, 0)
    start(1, 1)

    def pair(k, _):
        it0 = k * 2
        finish(it0, 0)

        @pl.when(it0 + 2 < SSTEPS)
        def _():
            start(it0 + 2, 0)

        finish(it0 + 1, 1)

        @pl.when(it0 + 3 < SSTEPS)
        def _():
            start(it0 + 3, 1)

        return ()

    lax.fori_loop(0, SSTEPS // 2, pair, ())
    if SSTEPS % 2 == 1:
        finish(SSTEPS - 1, 0)
    plsc.subcore_barrier()
    pltpu.sync_copy(acc_sh.at[pl.ds(s * NROWS_PER_TILE, NROWS_PER_TILE)],
                    out_hbm.at[pl.ds(c * NPAD + s * NROWS_PER_TILE,
                                     NROWS_PER_TILE)])


@functools.partial(
    pl.kernel,
    out_type=jax.ShapeDtypeStruct((2 * NPAD, D), jnp.float32),
    mesh=plsc.VectorSubcoreMesh(core_axis_name="c", subcore_axis_name="s"),
    scratch_types=[
        pltpu.VMEM((SSTEPS, SCHUNK), jnp.int32),
        pltpu.VMEM((SCHUNK, D), jnp.float32),
        pltpu.VMEM((SCHUNK, D), jnp.float32),
        pltpu.VMEM_SHARED((NPAD, D), jnp.float32),
        pltpu.SemaphoreType.DMA,
        pltpu.SemaphoreType.DMA,
    ],
)
def _sc_scatter(edges_hbm, dst_hbm, zs_hbm, out_hbm, idx_v, r0, r1, acc_sh,
                s0, s1):
    _sc_scatter_body(edges_hbm, dst_hbm, zs_hbm, out_hbm, idx_v, r0, r1, acc_sh,
                     s0, s1)


def _edge_body(g_ref, ea_ref, we_ref, b1_ref, w2_ref, b2_ref, ge_ref,
               bbe_ref, out_ref):
    ea = ea_ref[...]
    h = (g_ref[...]
         + jnp.dot(ea, we_ref[...], preferred_element_type=jnp.float32)
         + b1_ref[...])
    h = h * jax.nn.sigmoid(h)
    h = jnp.dot(h, w2_ref[...], preferred_element_type=jnp.float32) + b2_ref[...]
    mu = jnp.mean(h, axis=-1, keepdims=True)
    var = jnp.mean((h - mu) * (h - mu), axis=-1, keepdims=True)
    h = (h - mu) * jax.lax.rsqrt(var + 1e-5) * ge_ref[...] + bbe_ref[...]
    out_ref[...] = h + ea


def _edge_mlp(g, ea, we, b1, w2, b2, ge, bbe):
    grid = (E // ROW_BLK_EDGE,)
    row = lambda i: (i, 0)
    full = lambda i: (0, 0)
    return pl.pallas_call(
        _edge_body,
        grid=grid,
        in_specs=[
            pl.BlockSpec((ROW_BLK_EDGE, D), row),
            pl.BlockSpec((ROW_BLK_EDGE, D), row),
            pl.BlockSpec((D, D), full),
            pl.BlockSpec((1, D), full),
            pl.BlockSpec((D, D), full),
            pl.BlockSpec((1, D), full),
            pl.BlockSpec((1, D), full),
            pl.BlockSpec((1, D), full),
        ],
        out_specs=pl.BlockSpec((ROW_BLK_EDGE, D), row),
        out_shape=jax.ShapeDtypeStruct((E, D), jnp.float32),
    )(g, ea, we, b1, w2, b2, ge, bbe)


def _node_body(x_ref, p0_ref, p1_ref, wa_ref, wb_ref, b1_ref, w2_ref, b2_ref,
               gn_ref, bbn_ref, out_ref):
    x = x_ref[...]
    agg = p0_ref[...] + p1_ref[...]
    h = (jnp.dot(x, wa_ref[...], preferred_element_type=jnp.float32)
         + jnp.dot(agg, wb_ref[...], preferred_element_type=jnp.float32)
         + b1_ref[...])
    h = h * jax.nn.sigmoid(h)
    h = jnp.dot(h, w2_ref[...], preferred_element_type=jnp.float32) + b2_ref[...]
    mu = jnp.mean(h, axis=-1, keepdims=True)
    var = jnp.mean((h - mu) * (h - mu), axis=-1, keepdims=True)
    h = (h - mu) * jax.lax.rsqrt(var + 1e-5) * gn_ref[...] + bbn_ref[...]
    out_ref[...] = h + x


def _node_mlp(x, p0, p1, wa, wb, b1, w2, b2, gn, bbn):
    grid = (N // ROW_BLK_NODE,)
    row = lambda i: (i, 0)
    full = lambda i: (0, 0)
    return pl.pallas_call(
        _node_body,
        grid=grid,
        in_specs=[
            pl.BlockSpec((ROW_BLK_NODE, D), row),
            pl.BlockSpec((ROW_BLK_NODE, D), row),
            pl.BlockSpec((ROW_BLK_NODE, D), row),
            pl.BlockSpec((D, D), full),
            pl.BlockSpec((D, D), full),
            pl.BlockSpec((1, D), full),
            pl.BlockSpec((D, D), full),
            pl.BlockSpec((1, D), full),
            pl.BlockSpec((1, D), full),
            pl.BlockSpec((1, D), full),
        ],
        out_specs=pl.BlockSpec((ROW_BLK_NODE, D), row),
        out_shape=jax.ShapeDtypeStruct((N, D), jnp.float32),
    )(x, p0, p1, wa, wb, b1, w2, b2, gn, bbn)


def kernel(x, edge_attr, edge_index, shapes, we1, be1, we2, be2, ge, bbe,
           wn1, bn1, wn2, bn2, gn, bbn):
    del shapes
    src = edge_index[0]
    dst = edge_index[1]
    wi, wj, we = we1[:D], we1[D:2 * D], we1[2 * D:]
    px, qx = _precompute(x, wi, wj)
    dst3 = dst.reshape(SC_NW, GSTEPS, GCHUNK)
    src3 = src.reshape(SC_NW, GSTEPS, GCHUNK)
    g = _sc_gather(px, qx, dst3, src3)
    edges_new = _edge_mlp(g, edge_attr, we, be1[None, :], we2, be2[None, :],
                          ge[None, :], bbe[None, :])
    zs = jnp.zeros((NROWS_PER_TILE, D), jnp.float32)
    partials = _sc_scatter(edges_new, dst3, zs)
    nodes_new = _node_mlp(x, partials[:N], partials[NPAD:NPAD + N], wn1[:D],
                          wn1[D:], bn1[None, :], wn2, bn2[None, :], gn[None, :],
                          bbn[None, :])
    return nodes_new, edges_new


# speedup vs baseline: 5.8754x; 1.0001x over previous
import functools

import jax
import jax.numpy as jnp
from jax import lax
from jax.experimental import pallas as pl
from jax.experimental.pallas import tpu as pltpu
from jax.experimental.pallas import tpu_sc as plsc

N = 10000
E = 320000
D = 128

SC_NC = 2
SC_NS = 16
SC_NW = SC_NC * SC_NS
E_PER_W = E // SC_NW
GCHUNK = 80
GSTEPS = E_PER_W // GCHUNK

ROW_BLK_PRE = 2000
ROW_BLK_EDGE = 6400
ROW_BLK_NODE = 2000


def _pre_body(x_ref, wi_ref, wj_ref, px_ref, qx_ref):
    x = x_ref[...]
    px_ref[...] = jnp.dot(x, wi_ref[...], preferred_element_type=jnp.float32)
    qx_ref[...] = jnp.dot(x, wj_ref[...], preferred_element_type=jnp.float32)


def _precompute(x, wi, wj):
    grid = (N // ROW_BLK_PRE,)
    return pl.pallas_call(
        _pre_body,
        grid=grid,
        in_specs=[
            pl.BlockSpec((ROW_BLK_PRE, D), lambda i: (i, 0)),
            pl.BlockSpec((D, D), lambda i: (0, 0)),
            pl.BlockSpec((D, D), lambda i: (0, 0)),
        ],
        out_specs=[
            pl.BlockSpec((ROW_BLK_PRE, D), lambda i: (i, 0)),
            pl.BlockSpec((ROW_BLK_PRE, D), lambda i: (i, 0)),
        ],
        out_shape=[
            jax.ShapeDtypeStruct((N, D), jnp.float32),
            jax.ShapeDtypeStruct((N, D), jnp.float32),
        ],
    )(x, wi, wj)


def _sc_gather_body(px_hbm, qx_hbm, dst_hbm, src_hbm, g_hbm,
                    di_v, si_v, ra0, ra1, rb0, rb1, rc0, rc1,
                    sa0, sa1, sb0, sb1, sc0, sc1):
    wid = lax.axis_index("c") * SC_NS + lax.axis_index("s")
    wbase = wid * E_PER_W
    ras, rbs, rcs = (ra0, ra1), (rb0, rb1), (rc0, rc1)
    sas, sbs, scs = (sa0, sa1), (sb0, sb1), (sc0, sc1)
    pltpu.sync_copy(dst_hbm.at[wid], di_v)
    pltpu.sync_copy(src_hbm.at[wid], si_v)

    def start(it, b):
        pltpu.async_copy(px_hbm.at[di_v.at[it]], ras[b], sas[b])
        pltpu.async_copy(qx_hbm.at[si_v.at[it]], rbs[b], sbs[b])

    def wb_wait(it, b):
        base = wbase + it * GCHUNK
        pltpu.make_async_copy(rcs[b], g_hbm.at[pl.ds(base, GCHUNK)],
                              scs[b]).wait()

    def finish(it, b):
        pltpu.make_async_copy(px_hbm.at[di_v.at[it]], ras[b], sas[b]).wait()
        pltpu.make_async_copy(qx_hbm.at[si_v.at[it]], rbs[b], sbs[b]).wait()

        @pl.when(it >= 2)
        def _():
            wb_wait(it - 2, b)

        ra, rb, rc = ras[b], rbs[b], rcs[b]

        def add_row(r, _):
            for k in range(D // 16):
                sl = pl.ds(k * 16, 16)
                rc[r, sl] = ra[r, sl] + rb[r, sl]
            return ()

        lax.fori_loop(0, GCHUNK, add_row, ())
        base = wbase + it * GCHUNK
        pltpu.async_copy(rc, g_hbm.at[pl.ds(base, GCHUNK)], scs[b])

    start(0, 0)
    start(1, 1)

    def pair(k, _):
        it0 = k * 2
        finish(it0, 0)

        @pl.when(it0 + 2 < GSTEPS)
        def _():
            start(it0 + 2, 0)

        finish(it0 + 1, 1)

        @pl.when(it0 + 3 < GSTEPS)
        def _():
            start(it0 + 3, 1)

        return ()

    lax.fori_loop(0, GSTEPS // 2, pair, ())
    if GSTEPS % 2 == 1:
        finish(GSTEPS - 1, 0)
        wb_wait(GSTEPS - 2, 1)
        wb_wait(GSTEPS - 1, 0)
    else:
        wb_wait(GSTEPS - 2, 0)
        wb_wait(GSTEPS - 1, 1)


@functools.partial(
    pl.kernel,
    out_type=jax.ShapeDtypeStruct((E, D), jnp.float32),
    mesh=plsc.VectorSubcoreMesh(core_axis_name="c", subcore_axis_name="s"),
    scratch_types=[
        pltpu.VMEM((GSTEPS, GCHUNK), jnp.int32),
        pltpu.VMEM((GSTEPS, GCHUNK), jnp.int32),
        pltpu.VMEM((GCHUNK, D), jnp.float32),
        pltpu.VMEM((GCHUNK, D), jnp.float32),
        pltpu.VMEM((GCHUNK, D), jnp.float32),
        pltpu.VMEM((GCHUNK, D), jnp.float32),
        pltpu.VMEM((GCHUNK, D), jnp.float32),
        pltpu.VMEM((GCHUNK, D), jnp.float32),
        pltpu.SemaphoreType.DMA,
        pltpu.SemaphoreType.DMA,
        pltpu.SemaphoreType.DMA,
        pltpu.SemaphoreType.DMA,
        pltpu.SemaphoreType.DMA,
        pltpu.SemaphoreType.DMA,
    ],
)
def _sc_gather(px_hbm, qx_hbm, dst_hbm, src_hbm, g_hbm,
               di_v, si_v, ra0, ra1, rb0, rb1, rc0, rc1,
               sa0, sa1, sb0, sb1, sc0, sc1):
    _sc_gather_body(px_hbm, qx_hbm, dst_hbm, src_hbm, g_hbm,
                    di_v, si_v, ra0, ra1, rb0, rb1, rc0, rc1,
                    sa0, sa1, sb0, sb1, sc0, sc1)


NPAD = 10240
NROWS_PER_TILE = NPAD // SC_NS
SCHUNK = 80
SSTEPS = E_PER_W // SCHUNK


def _sc_scatter_body(edges_hbm, dst_hbm, zs_hbm, out_hbm, idx_v, r0, r1, acc_sh,
                     s0, s1, t0, t1):
    c = lax.axis_index("c")
    s = lax.axis_index("s")
    wid = c * SC_NS + s
    wbase = wid * E_PER_W
    rows, sems, asems = (r0, r1), (s0, s1), (t0, t1)
    pltpu.sync_copy(zs_hbm, acc_sh.at[pl.ds(s * NROWS_PER_TILE, NROWS_PER_TILE)])
    pltpu.sync_copy(dst_hbm.at[wid], idx_v)
    plsc.subcore_barrier()

    def add_wait(it, b):
        pltpu.make_async_copy(rows[b], acc_sh.at[idx_v.at[it]], asems[b]).wait()

    def start(it, b):
        @pl.when(it >= 2)
        def _():
            add_wait(it - 2, b)

        base = wbase + it * SCHUNK
        pltpu.async_copy(edges_hbm.at[pl.ds(base, SCHUNK)], rows[b], sems[b])

    def finish(it, b):
        base = wbase + it * SCHUNK
        pltpu.make_async_copy(edges_hbm.at[pl.ds(base, SCHUNK)], rows[b],
                              sems[b]).wait()
        pltpu.async_copy(rows[b], acc_sh.at[idx_v.at[it]], asems[b], add=True)

    start(0, 0)
    start(1, 1)

    def pair(k, _):
        it0 = k * 2
        finish(it0, 0)

        @pl.when(it0 + 2 < SSTEPS)
        def _():
            start(it0 + 2, 0)

        finish(it0 + 1, 1)

        @pl.when(it0 + 3 < SSTEPS)
        def _():
            start(it0 + 3, 1)

        return ()

    lax.fori_loop(0, SSTEPS // 2, pair, ())
    if SSTEPS % 2 == 1:
        finish(SSTEPS - 1, 0)
        add_wait(SSTEPS - 2, 1)
        add_wait(SSTEPS - 1, 0)
    else:
        add_wait(SSTEPS - 2, 0)
        add_wait(SSTEPS - 1, 1)
    plsc.subcore_barrier()
    pltpu.sync_copy(acc_sh.at[pl.ds(s * NROWS_PER_TILE, NROWS_PER_TILE)],
                    out_hbm.at[pl.ds(c * NPAD + s * NROWS_PER_TILE,
                                     NROWS_PER_TILE)])


@functools.partial(
    pl.kernel,
    out_type=jax.ShapeDtypeStruct((2 * NPAD, D), jnp.float32),
    mesh=plsc.VectorSubcoreMesh(core_axis_name="c", subcore_axis_name="s"),
    scratch_types=[
        pltpu.VMEM((SSTEPS, SCHUNK), jnp.int32),
        pltpu.VMEM((SCHUNK, D), jnp.float32),
        pltpu.VMEM((SCHUNK, D), jnp.float32),
        pltpu.VMEM_SHARED((NPAD, D), jnp.float32),
        pltpu.SemaphoreType.DMA,
        pltpu.SemaphoreType.DMA,
        pltpu.SemaphoreType.DMA,
        pltpu.SemaphoreType.DMA,
    ],
)
def _sc_scatter(edges_hbm, dst_hbm, zs_hbm, out_hbm, idx_v, r0, r1, acc_sh,
                s0, s1, t0, t1):
    _sc_scatter_body(edges_hbm, dst_hbm, zs_hbm, out_hbm, idx_v, r0, r1, acc_sh,
                     s0, s1, t0, t1)


def _edge_body(g_ref, ea_ref, we_ref, b1_ref, w2_ref, b2_ref, ge_ref,
               bbe_ref, out_ref):
    ea = ea_ref[...]
    h = (g_ref[...]
         + jnp.dot(ea, we_ref[...], preferred_element_type=jnp.float32)
         + b1_ref[...])
    h = h * jax.nn.sigmoid(h)
    h = jnp.dot(h, w2_ref[...], preferred_element_type=jnp.float32) + b2_ref[...]
    mu = jnp.mean(h, axis=-1, keepdims=True)
    var = jnp.mean((h - mu) * (h - mu), axis=-1, keepdims=True)
    h = (h - mu) * jax.lax.rsqrt(var + 1e-5) * ge_ref[...] + bbe_ref[...]
    out_ref[...] = h + ea


def _edge_mlp(g, ea, we, b1, w2, b2, ge, bbe):
    grid = (E // ROW_BLK_EDGE,)
    row = lambda i: (i, 0)
    full = lambda i: (0, 0)
    return pl.pallas_call(
        _edge_body,
        grid=grid,
        in_specs=[
            pl.BlockSpec((ROW_BLK_EDGE, D), row),
            pl.BlockSpec((ROW_BLK_EDGE, D), row),
            pl.BlockSpec((D, D), full),
            pl.BlockSpec((1, D), full),
            pl.BlockSpec((D, D), full),
            pl.BlockSpec((1, D), full),
            pl.BlockSpec((1, D), full),
            pl.BlockSpec((1, D), full),
        ],
        out_specs=pl.BlockSpec((ROW_BLK_EDGE, D), row),
        out_shape=jax.ShapeDtypeStruct((E, D), jnp.float32),
    )(g, ea, we, b1, w2, b2, ge, bbe)


def _node_body(x_ref, p0_ref, p1_ref, wa_ref, wb_ref, b1_ref, w2_ref, b2_ref,
               gn_ref, bbn_ref, out_ref):
    x = x_ref[...]
    agg = p0_ref[...] + p1_ref[...]
    h = (jnp.dot(x, wa_ref[...], preferred_element_type=jnp.float32)
         + jnp.dot(agg, wb_ref[...], preferred_element_type=jnp.float32)
         + b1_ref[...])
    h = h * jax.nn.sigmoid(h)
    h = jnp.dot(h, w2_ref[...], preferred_element_type=jnp.float32) + b2_ref[...]
    mu = jnp.mean(h, axis=-1, keepdims=True)
    var = jnp.mean((h - mu) * (h - mu), axis=-1, keepdims=True)
    h = (h - mu) * jax.lax.rsqrt(var + 1e-5) * gn_ref[...] + bbn_ref[...]
    out_ref[...] = h + x


def _node_mlp(x, p0, p1, wa, wb, b1, w2, b2, gn, bbn):
    grid = (N // ROW_BLK_NODE,)
    row = lambda i: (i, 0)
    full = lambda i: (0, 0)
    return pl.pallas_call(
        _node_body,
        grid=grid,
        in_specs=[
            pl.BlockSpec((ROW_BLK_NODE, D), row),
            pl.BlockSpec((ROW_BLK_NODE, D), row),
            pl.BlockSpec((ROW_BLK_NODE, D), row),
            pl.BlockSpec((D, D), full),
            pl.BlockSpec((D, D), full),
            pl.BlockSpec((1, D), full),
            pl.BlockSpec((D, D), full),
            pl.BlockSpec((1, D), full),
            pl.BlockSpec((1, D), full),
            pl.BlockSpec((1, D), full),
        ],
        out_specs=pl.BlockSpec((ROW_BLK_NODE, D), row),
        out_shape=jax.ShapeDtypeStruct((N, D), jnp.float32),
    )(x, p0, p1, wa, wb, b1, w2, b2, gn, bbn)


def kernel(x, edge_attr, edge_index, shapes, we1, be1, we2, be2, ge, bbe,
           wn1, bn1, wn2, bn2, gn, bbn):
    del shapes
    src = edge_index[0]
    dst = edge_index[1]
    wi, wj, we = we1[:D], we1[D:2 * D], we1[2 * D:]
    px, qx = _precompute(x, wi, wj)
    dst3 = dst.reshape(SC_NW, GSTEPS, GCHUNK)
    src3 = src.reshape(SC_NW, GSTEPS, GCHUNK)
    g = _sc_gather(px, qx, dst3, src3)
    edges_new = _edge_mlp(g, edge_attr, we, be1[None, :], we2, be2[None, :],
                          ge[None, :], bbe[None, :])
    zs = jnp.zeros((NROWS_PER_TILE, D), jnp.float32)
    partials = _sc_scatter(edges_new, dst3, zs)
    nodes_new = _node_mlp(x, partials[:N], partials[NPAD:NPAD + N], wn1[:D],
                          wn1[D:], bn1[None, :], wn2, bn2[None, :], gn[None, :],
                          bbn[None, :])
    return nodes_new, edges_new
